# baseline (speedup 1.0000x reference)
import functools
import math

import jax
import jax.numpy as jnp
from jax import lax
from jax.experimental import pallas as pl
from jax.experimental.pallas import tpu as pltpu
from jax.experimental.pallas import tpu_sc as plsc

_LANES = 16


def _combo_tc_body(seg_ref, out_ref):
    R, D = out_ref.shape
    r = lax.broadcasted_iota(jnp.int32, (R, D), 0)
    dcol = lax.broadcasted_iota(jnp.int32, (R, D), 1)
    l3 = r // 3
    s = r - 3 * l3
    half = (dcol // 2).astype(jnp.float32)
    div = jnp.exp(half * (-2.0 * math.log(10000.0) / D))
    ang = l3.astype(jnp.float32) * div
    pe = jnp.where(dcol % 2 == 0, jnp.sin(ang), jnp.cos(ang))
    st = seg_ref[...]
    seg0 = jnp.broadcast_to(st[0:1, :], (R, D))
    seg1 = jnp.broadcast_to(st[1:2, :], (R, D))
    seg2 = jnp.broadcast_to(st[2:3, :], (R, D))
    out_ref[...] = pe + jnp.where(s == 0, seg0, jnp.where(s == 1, seg1, seg2))


def _build_combo(segment_table, rows):
    return pl.pallas_call(
        _combo_tc_body,
        out_shape=jax.ShapeDtypeStruct((rows, segment_table.shape[1]), jnp.float32),
    )(segment_table)


def _sc_lookup(seq_flat, lbl_flat, token_table, combo, L):
    N = seq_flat.shape[0]
    D = token_table.shape[1]
    info = plsc.get_sparse_core_info()
    NC, NS = info.num_cores, info.num_subcores
    NW = NC * NS
    C = 128
    assert N % (NW * C) == 0 and D % _LANES == 0
    rows_per_w = N // NW
    chunks = rows_per_w // C
    assert rows_per_w % L == 0 and C <= L
    assert chunks % 2 == 0 and chunks >= 4
    mesh = plsc.VectorSubcoreMesh(core_axis_name="c", subcore_axis_name="s")

    @functools.partial(
        pl.kernel,
        out_type=jax.ShapeDtypeStruct((N, D), jnp.float32),
        mesh=mesh,
        scratch_types=[
            pltpu.VMEM((C,), jnp.int32), pltpu.VMEM((C,), jnp.int32),
            pltpu.VMEM((C,), jnp.int32), pltpu.VMEM((C,), jnp.int32),
            pltpu.VMEM((C, D), jnp.float32), pltpu.VMEM((C, D), jnp.float32),
            pltpu.VMEM((C, D), jnp.float32), pltpu.VMEM((C, D), jnp.float32),
            pltpu.VMEM((C, D), jnp.float32), pltpu.VMEM((C, D), jnp.float32),
            pltpu.SemaphoreType.DMA, pltpu.SemaphoreType.DMA,
            pltpu.SemaphoreType.DMA, pltpu.SemaphoreType.DMA,
            pltpu.SemaphoreType.DMA, pltpu.SemaphoreType.DMA,
        ],
    )
    def k(seq_hbm, lbl_hbm, tok_hbm, combo_hbm, out_hbm, *sc):
        sidx, cidx = (sc[0], sc[1]), (sc[2], sc[3])
        tok, cmb, outv = (sc[4], sc[5]), (sc[6], sc[7]), (sc[8], sc[9])
        tsem, csem, wsem = (sc[10], sc[11]), (sc[12], sc[13]), (sc[14], sc[15])
        wid = lax.axis_index("s") * NC + lax.axis_index("c")
        tile_base = wid * rows_per_w

        def fire(ci, b, lpos0):
            base = tile_base + ci * C
            pltpu.sync_copy(seq_hbm.at[pl.ds(base, C)], sidx[b])
            pltpu.sync_copy(lbl_hbm.at[pl.ds(base, C)], cidx[b])
            for j in range(C // _LANES):
                v = lpos0 + (j * _LANES + lax.iota(jnp.int32, _LANES))
                lpos = jnp.where(v >= L, v - L, v)
                sl = pl.ds(j * _LANES, _LANES)
                cidx[b][sl] = 3 * lpos + cidx[b][sl]
            pltpu.async_copy(tok_hbm.at[sidx[b]], tok[b], tsem[b])
            pltpu.async_copy(combo_hbm.at[cidx[b]], cmb[b], csem[b])

        def advance2(v):
            w = v + 2 * C
            for _ in range((2 * C) // L + 1):
                w = jnp.where(w >= L, w - L, w)
            return w

        fire(jnp.int32(0), 0, jnp.int32(0))
        fire(jnp.int32(1), 1, jnp.int32(C % L))

        def step(kk, ci, b, lp):
            base = tile_base + ci * C
            pltpu.make_async_copy(tok_hbm.at[sidx[b]], tok[b], tsem[b]).wait()
            pltpu.make_async_copy(combo_hbm.at[cidx[b]], cmb[b], csem[b]).wait()

            @pl.when(kk > 0)
            def _():
                pltpu.make_async_copy(outv[b], out_hbm.at[pl.ds(base, C)],
                                      wsem[b]).wait()

            def addrow(r, cc):
                for u in range(D // _LANES):
                    sl = pl.ds(u * _LANES, _LANES)
                    outv[b][r, sl] = tok[b][r, sl] + cmb[b][r, sl]
                return cc

            lax.fori_loop(0, C, addrow, 0, unroll=2)
            pltpu.async_copy(outv[b], out_hbm.at[pl.ds(base, C)], wsem[b])

            @pl.when(ci + 2 < chunks)
            def _():
                fire(ci + 2, b, lp)

        def body(kk, carry):
            lp0, lp1 = carry
            step(kk, 2 * kk, 0, lp0)
            step(kk, 2 * kk + 1, 1, lp1)
            return (advance2(lp0), advance2(lp1))

        lax.fori_loop(0, chunks // 2, body,
                      (jnp.int32((2 * C) % L), jnp.int32((3 * C) % L)),
                      unroll=False)
        pltpu.make_async_copy(outv[0], out_hbm.at[pl.ds(tile_base, C)],
                              wsem[0]).wait()
        pltpu.make_async_copy(outv[1], out_hbm.at[pl.ds(tile_base, C)],
                              wsem[1]).wait()

    return k(seq_flat, lbl_flat, token_table, combo)


def kernel(sequence, segment_label, token_table, segment_table):
    B, L = sequence.shape
    D = token_table.shape[1]
    combo_rows = ((3 * L + 7) // 8) * 8
    combo = _build_combo(segment_table, combo_rows)
    seq_flat = sequence.reshape(-1).astype(jnp.int32)
    lbl_flat = segment_label.reshape(-1).astype(jnp.int32)
    out = _sc_lookup(seq_flat, lbl_flat, token_table, combo, L)
    return out.reshape(B, L, D)


# speedup vs baseline: 1.4638x; 1.4638x over previous
import functools
import math

import jax
import jax.numpy as jnp
from jax import lax
from jax.experimental import pallas as pl
from jax.experimental.pallas import tpu as pltpu
from jax.experimental.pallas import tpu_sc as plsc

_LANES = 16


def _combo_tc_body(seg_ref, out_ref):
    R, D = out_ref.shape
    r = lax.broadcasted_iota(jnp.int32, (R, D), 0)
    dcol = lax.broadcasted_iota(jnp.int32, (R, D), 1)
    l3 = r // 3
    s = r - 3 * l3
    half = (dcol // 2).astype(jnp.float32)
    div = jnp.exp(half * (-2.0 * math.log(10000.0) / D))
    ang = l3.astype(jnp.float32) * div
    pe = jnp.where(dcol % 2 == 0, jnp.sin(ang), jnp.cos(ang))
    st = seg_ref[...]
    seg0 = jnp.broadcast_to(st[0:1, :], (R, D))
    seg1 = jnp.broadcast_to(st[1:2, :], (R, D))
    seg2 = jnp.broadcast_to(st[2:3, :], (R, D))
    out_ref[...] = pe + jnp.where(s == 0, seg0, jnp.where(s == 1, seg1, seg2))


def _build_combo(segment_table, rows):
    return pl.pallas_call(
        _combo_tc_body,
        out_shape=jax.ShapeDtypeStruct((rows, segment_table.shape[1]), jnp.float32),
    )(segment_table)


def _sc_lookup(seq_flat, lbl_flat, token_table, combo, L):
    N = seq_flat.shape[0]
    D = token_table.shape[1]
    info = plsc.get_sparse_core_info()
    NC, NS = info.num_cores, info.num_subcores
    NW = NC * NS
    C = 128
    assert N % (NW * C) == 0 and D % _LANES == 0
    rows_per_w = N // NW
    chunks = rows_per_w // C
    assert rows_per_w % L == 0 and C <= L
    mesh = plsc.VectorSubcoreMesh(core_axis_name="c", subcore_axis_name="s")

    @functools.partial(
        pl.kernel,
        out_type=jax.ShapeDtypeStruct((N, D), jnp.float32),
        mesh=mesh,
        scratch_types=[
            pltpu.VMEM((C,), jnp.int32),
            pltpu.VMEM((C,), jnp.int32),
            pltpu.VMEM((C, D), jnp.float32),
            pltpu.VMEM((C, D), jnp.float32),
            pltpu.VMEM_SHARED((NS * ((3 * L + NS * 8 - 1) // (NS * 8)) * 8, D),
                              jnp.float32),
            pltpu.SemaphoreType.DMA,
            pltpu.SemaphoreType.DMA,
        ],
    )
    def k(seq_hbm, lbl_hbm, tok_hbm, combo_hbm, out_hbm,
          sidx_v, cidx_v, tok_v, cmb_v, combo_spm, sem_t, sem_c):
        wid = lax.axis_index("s") * NC + lax.axis_index("c")
        tile_base = wid * rows_per_w
        sid = lax.axis_index("s")
        rows_per_tile = combo_spm.shape[0] // NS
        pltpu.sync_copy(combo_hbm.at[pl.ds(sid * rows_per_tile, rows_per_tile)],
                        combo_spm.at[pl.ds(sid * rows_per_tile, rows_per_tile)])
        plsc.subcore_barrier()

        def chunk(c, lpos0):
            base = tile_base + c * C
            pltpu.sync_copy(seq_hbm.at[pl.ds(base, C)], sidx_v)
            pltpu.sync_copy(lbl_hbm.at[pl.ds(base, C)], cidx_v)
            tok_dma = pltpu.async_copy(tok_hbm.at[sidx_v], tok_v, sem_t)
            for j in range(C // _LANES):
                v = lpos0 + (j * _LANES + lax.iota(jnp.int32, _LANES))
                lpos = jnp.where(v >= L, v - L, v)
                sl = pl.ds(j * _LANES, _LANES)
                cidx_v[sl] = 3 * lpos + cidx_v[sl]
            cmb_dma = pltpu.async_copy(combo_spm.at[cidx_v], cmb_v, sem_c)
            tok_dma.wait()
            cmb_dma.wait()

            def addrow(r, carry2):
                for u in range(D // _LANES):
                    sl = pl.ds(u * _LANES, _LANES)
                    tok_v[r, sl] = tok_v[r, sl] + cmb_v[r, sl]
                return carry2

            lax.fori_loop(0, C, addrow, 0, unroll=False)
            pltpu.sync_copy(tok_v, out_hbm.at[pl.ds(base, C)])
            nxt = lpos0 + C
            return jnp.where(nxt >= L, nxt - L, nxt)

        lax.fori_loop(0, chunks, chunk, jnp.int32(0), unroll=False)

    return k(seq_flat, lbl_flat, token_table, combo)


def kernel(sequence, segment_label, token_table, segment_table):
    B, L = sequence.shape
    D = token_table.shape[1]
    NS = plsc.get_sparse_core_info().num_subcores
    combo_rows = NS * ((3 * L + NS * 8 - 1) // (NS * 8)) * 8
    combo = _build_combo(segment_table, combo_rows)
    seq_flat = sequence.reshape(-1).astype(jnp.int32)
    lbl_flat = segment_label.reshape(-1).astype(jnp.int32)
    out = _sc_lookup(seq_flat, lbl_flat, token_table, combo, L)
    return out.reshape(B, L, D)


# speedup vs baseline: 2.0707x; 1.4146x over previous
import functools
import math

import jax
import jax.numpy as jnp
from jax import lax
from jax.experimental import pallas as pl
from jax.experimental.pallas import tpu as pltpu
from jax.experimental.pallas import tpu_sc as plsc

_LANES = 16


def _combo_tc_body(seg_ref, out_ref):
    R, D = out_ref.shape
    r = lax.broadcasted_iota(jnp.int32, (R, D), 0)
    dcol = lax.broadcasted_iota(jnp.int32, (R, D), 1)
    l3 = r // 3
    s = r - 3 * l3
    half = (dcol // 2).astype(jnp.float32)
    div = jnp.exp(half * (-2.0 * math.log(10000.0) / D))
    ang = l3.astype(jnp.float32) * div
    pe = jnp.where(dcol % 2 == 0, jnp.sin(ang), jnp.cos(ang))
    st = seg_ref[...]
    seg0 = jnp.broadcast_to(st[0:1, :], (R, D))
    seg1 = jnp.broadcast_to(st[1:2, :], (R, D))
    seg2 = jnp.broadcast_to(st[2:3, :], (R, D))
    out_ref[...] = pe + jnp.where(s == 0, seg0, jnp.where(s == 1, seg1, seg2))


def _build_combo(segment_table, rows):
    return pl.pallas_call(
        _combo_tc_body,
        out_shape=jax.ShapeDtypeStruct((rows, segment_table.shape[1]), jnp.float32),
    )(segment_table)


def _sc_lookup(seq_flat, lbl_flat, token_table, combo, L):
    N = seq_flat.shape[0]
    D = token_table.shape[1]
    info = plsc.get_sparse_core_info()
    NC, NS = info.num_cores, info.num_subcores
    NW = NC * NS
    C = 128
    assert N % (NW * C) == 0 and D % _LANES == 0
    rows_per_w = N // NW
    chunks = rows_per_w // C
    assert rows_per_w % L == 0 and C <= L
    mesh = plsc.VectorSubcoreMesh(core_axis_name="c", subcore_axis_name="s")

    @functools.partial(
        pl.kernel,
        out_type=jax.ShapeDtypeStruct((N, D), jnp.float32),
        mesh=mesh,
        scratch_types=[
            pltpu.VMEM((C,), jnp.int32), pltpu.VMEM((C,), jnp.int32),
            pltpu.VMEM((C,), jnp.int32), pltpu.VMEM((C,), jnp.int32),
            pltpu.VMEM((C, D), jnp.float32), pltpu.VMEM((C, D), jnp.float32),
            pltpu.VMEM((C, D), jnp.float32), pltpu.VMEM((C, D), jnp.float32),
            pltpu.VMEM_SHARED((NS * ((3 * L + NS * 8 - 1) // (NS * 8)) * 8, D),
                              jnp.float32),
            pltpu.SemaphoreType.DMA, pltpu.SemaphoreType.DMA,
            pltpu.SemaphoreType.DMA, pltpu.SemaphoreType.DMA,
            pltpu.SemaphoreType.DMA, pltpu.SemaphoreType.DMA,
        ],
    )
    def k(seq_hbm, lbl_hbm, tok_hbm, combo_hbm, out_hbm, *sc):
        sidx, cidx = (sc[0], sc[1]), (sc[2], sc[3])
        tok, cmb = (sc[4], sc[5]), (sc[6], sc[7])
        combo_spm = sc[8]
        tsem, csem, wsem = (sc[9], sc[10]), (sc[11], sc[12]), (sc[13], sc[14])
        wid = lax.axis_index("s") * NC + lax.axis_index("c")
        tile_base = wid * rows_per_w
        sid = lax.axis_index("s")
        rows_per_tile = combo_spm.shape[0] // NS
        pltpu.sync_copy(combo_hbm.at[pl.ds(sid * rows_per_tile, rows_per_tile)],
                        combo_spm.at[pl.ds(sid * rows_per_tile, rows_per_tile)])
        plsc.subcore_barrier()

        def fire(ci, b, lpos0):
            base = tile_base + ci * C
            pltpu.sync_copy(seq_hbm.at[pl.ds(base, C)], sidx[b])
            pltpu.sync_copy(lbl_hbm.at[pl.ds(base, C)], cidx[b])
            tg = pltpu.async_copy(tok_hbm.at[sidx[b]], tok[b], tsem[b])
            for j in range(C // _LANES):
                v = lpos0 + (j * _LANES + lax.iota(jnp.int32, _LANES))
                lpos = jnp.where(v >= L, v - L, v)
                sl = pl.ds(j * _LANES, _LANES)
                cidx[b][sl] = 3 * lpos + cidx[b][sl]
            cg = pltpu.async_copy(combo_spm.at[cidx[b]], cmb[b], csem[b])
            return tg, cg

        def addwb(ci, b, tg, cg):
            base = tile_base + ci * C
            tg.wait()
            cg.wait()

            def addrow(r, carry2):
                for u in range(D // _LANES):
                    sl = pl.ds(u * _LANES, _LANES)
                    tok[b][r, sl] = tok[b][r, sl] + cmb[b][r, sl]
                return carry2

            lax.fori_loop(0, C, addrow, 0, unroll=False)
            return pltpu.async_copy(tok[b], out_hbm.at[pl.ds(base, C)], wsem[b])

        def pair(kk, lpos0):
            c0 = 2 * kk
            lpos1 = lpos0 + C
            lpos1 = jnp.where(lpos1 >= L, lpos1 - L, lpos1)
            tg0, cg0 = fire(c0, 0, lpos0)
            tg1, cg1 = fire(c0 + 1, 1, lpos1)
            wb0 = addwb(c0, 0, tg0, cg0)
            wb1 = addwb(c0 + 1, 1, tg1, cg1)
            wb0.wait()
            wb1.wait()
            nxt = lpos1 + C
            return jnp.where(nxt >= L, nxt - L, nxt)

        lax.fori_loop(0, chunks // 2, pair, jnp.int32(0), unroll=False)

    return k(seq_flat, lbl_flat, token_table, combo)


def kernel(sequence, segment_label, token_table, segment_table):
    B, L = sequence.shape
    D = token_table.shape[1]
    NS = plsc.get_sparse_core_info().num_subcores
    combo_rows = NS * ((3 * L + NS * 8 - 1) // (NS * 8)) * 8
    combo = _build_combo(segment_table, combo_rows)
    seq_flat = sequence.reshape(-1).astype(jnp.int32)
    lbl_flat = segment_label.reshape(-1).astype(jnp.int32)
    out = _sc_lookup(seq_flat, lbl_flat, token_table, combo, L)
    return out.reshape(B, L, D)


# speedup vs baseline: 2.1662x; 1.0461x over previous
import functools
import math

import jax
import jax.numpy as jnp
from jax import lax
from jax.experimental import pallas as pl
from jax.experimental.pallas import tpu as pltpu
from jax.experimental.pallas import tpu_sc as plsc

_LANES = 16


def _combo_tc_body(seg_ref, out_ref):
    R, D = out_ref.shape
    r = lax.broadcasted_iota(jnp.int32, (R, D), 0)
    dcol = lax.broadcasted_iota(jnp.int32, (R, D), 1)
    l3 = r // 3
    s = r - 3 * l3
    half = (dcol // 2).astype(jnp.float32)
    div = jnp.exp(half * (-2.0 * math.log(10000.0) / D))
    ang = l3.astype(jnp.float32) * div
    pe = jnp.where(dcol % 2 == 0, jnp.sin(ang), jnp.cos(ang))
    st = seg_ref[...]
    seg0 = jnp.broadcast_to(st[0:1, :], (R, D))
    seg1 = jnp.broadcast_to(st[1:2, :], (R, D))
    seg2 = jnp.broadcast_to(st[2:3, :], (R, D))
    out_ref[...] = pe + jnp.where(s == 0, seg0, jnp.where(s == 1, seg1, seg2))


def _build_combo(segment_table, rows):
    return pl.pallas_call(
        _combo_tc_body,
        out_shape=jax.ShapeDtypeStruct((rows, segment_table.shape[1]), jnp.float32),
    )(segment_table)


def _sc_lookup(seq_flat, lbl_flat, token_table, combo, L):
    N = seq_flat.shape[0]
    D = token_table.shape[1]
    info = plsc.get_sparse_core_info()
    NC, NS = info.num_cores, info.num_subcores
    NW = NC * NS
    C = 128
    assert N % (NW * C) == 0 and D % _LANES == 0
    rows_per_w = N // NW
    chunks = rows_per_w // C
    assert rows_per_w % L == 0 and C <= L
    mesh = plsc.VectorSubcoreMesh(core_axis_name="c", subcore_axis_name="s")

    @functools.partial(
        pl.kernel,
        out_type=jax.ShapeDtypeStruct((N, D), jnp.float32),
        mesh=mesh,
        scratch_types=[
            pltpu.VMEM((chunks, C), jnp.int32),
            pltpu.VMEM((chunks, C), jnp.int32),
            pltpu.VMEM((C, D), jnp.float32), pltpu.VMEM((C, D), jnp.float32),
            pltpu.VMEM((C, D), jnp.float32), pltpu.VMEM((C, D), jnp.float32),
            pltpu.VMEM_SHARED((NS * ((3 * L + NS * 8 - 1) // (NS * 8)) * 8, D),
                              jnp.float32),
            pltpu.SemaphoreType.DMA, pltpu.SemaphoreType.DMA,
            pltpu.SemaphoreType.DMA, pltpu.SemaphoreType.DMA,
            pltpu.SemaphoreType.DMA, pltpu.SemaphoreType.DMA,
        ],
    )
    def k(seq_hbm, lbl_hbm, tok_hbm, combo_hbm, out_hbm, *sc):
        sidx_all, cidx_all = sc[0], sc[1]
        tok, cmb = (sc[2], sc[3]), (sc[4], sc[5])
        combo_spm = sc[6]
        tsem, csem, wsem = (sc[7], sc[8]), (sc[9], sc[10]), (sc[11], sc[12])
        wid = lax.axis_index("s") * NC + lax.axis_index("c")
        tile_base = wid * rows_per_w
        sid = lax.axis_index("s")
        rows_per_tile = combo_spm.shape[0] // NS
        pltpu.sync_copy(combo_hbm.at[pl.ds(sid * rows_per_tile, rows_per_tile)],
                        combo_spm.at[pl.ds(sid * rows_per_tile, rows_per_tile)])
        pltpu.sync_copy(seq_hbm.at[wid], sidx_all)
        pltpu.sync_copy(lbl_hbm.at[wid], cidx_all)

        def cvt(c, lpos0):
            for j in range(C // _LANES):
                v = lpos0 + (j * _LANES + lax.iota(jnp.int32, _LANES))
                lpos = jnp.where(v >= L, v - L, v)
                sl = pl.ds(j * _LANES, _LANES)
                cidx_all[c, sl] = 3 * lpos + cidx_all[c, sl]
            nxt = lpos0 + C
            return jnp.where(nxt >= L, nxt - L, nxt)

        lax.fori_loop(0, chunks, cvt, jnp.int32(0), unroll=False)
        plsc.subcore_barrier()

        def fire(ci, b):
            tg = pltpu.async_copy(tok_hbm.at[sidx_all.at[ci]], tok[b], tsem[b])
            cg = pltpu.async_copy(combo_spm.at[cidx_all.at[ci]], cmb[b], csem[b])
            return tg, cg

        def addwb(ci, b, tg, cg):
            base = tile_base + ci * C
            tg.wait()
            cg.wait()

            def addrow(r, carry2):
                for u in range(D // _LANES):
                    sl = pl.ds(u * _LANES, _LANES)
                    tok[b][r, sl] = tok[b][r, sl] + cmb[b][r, sl]
                return carry2

            lax.fori_loop(0, C, addrow, 0, unroll=False)
            return pltpu.async_copy(tok[b], out_hbm.at[pl.ds(base, C)], wsem[b])

        def pair(kk, carry):
            c0 = 2 * kk
            tg0, cg0 = fire(c0, 0)
            tg1, cg1 = fire(c0 + 1, 1)
            wb0 = addwb(c0, 0, tg0, cg0)
            wb1 = addwb(c0 + 1, 1, tg1, cg1)
            wb0.wait()
            wb1.wait()
            return carry

        lax.fori_loop(0, chunks // 2, pair, jnp.int32(0), unroll=False)

    seq3 = seq_flat.reshape(NW, chunks, C)
    lbl3 = lbl_flat.reshape(NW, chunks, C)
    return k(seq3, lbl3, token_table, combo)


def kernel(sequence, segment_label, token_table, segment_table):
    B, L = sequence.shape
    D = token_table.shape[1]
    NS = plsc.get_sparse_core_info().num_subcores
    combo_rows = NS * ((3 * L + NS * 8 - 1) // (NS * 8)) * 8
    combo = _build_combo(segment_table, combo_rows)
    seq_flat = sequence.reshape(-1).astype(jnp.int32)
    lbl_flat = segment_label.reshape(-1).astype(jnp.int32)
    out = _sc_lookup(seq_flat, lbl_flat, token_table, combo, L)
    return out.reshape(B, L, D)


# speedup vs baseline: 2.8175x; 1.3007x over previous
import functools
import math

import jax
import jax.numpy as jnp
from jax import lax
from jax.experimental import pallas as pl
from jax.experimental.pallas import tpu as pltpu
from jax.experimental.pallas import tpu_sc as plsc

_LANES = 16


def _combo_tc_body(seg_ref, out_ref):
    R, D = out_ref.shape
    r = lax.broadcasted_iota(jnp.int32, (R, D), 0)
    dcol = lax.broadcasted_iota(jnp.int32, (R, D), 1)
    l3 = r // 3
    s = r - 3 * l3
    half = (dcol // 2).astype(jnp.float32)
    div = jnp.exp(half * (-2.0 * math.log(10000.0) / D))
    ang = l3.astype(jnp.float32) * div
    pe = jnp.where(dcol % 2 == 0, jnp.sin(ang), jnp.cos(ang))
    st = seg_ref[...]
    seg0 = jnp.broadcast_to(st[0:1, :], (R, D))
    seg1 = jnp.broadcast_to(st[1:2, :], (R, D))
    seg2 = jnp.broadcast_to(st[2:3, :], (R, D))
    out_ref[...] = pe + jnp.where(s == 0, seg0, jnp.where(s == 1, seg1, seg2))


def _build_combo(segment_table, rows):
    return pl.pallas_call(
        _combo_tc_body,
        out_shape=jax.ShapeDtypeStruct((rows, segment_table.shape[1]), jnp.float32),
    )(segment_table)


def _sc_lookup(seq_flat, lbl_flat, token_table, combo, L):
    N = seq_flat.shape[0]
    D = token_table.shape[1]
    info = plsc.get_sparse_core_info()
    NC, NS = info.num_cores, info.num_subcores
    NW = NC * NS
    C = 128
    assert N % (NW * C) == 0 and D % _LANES == 0
    rows_per_w = N // NW
    chunks = rows_per_w // C
    assert rows_per_w % L == 0 and C <= L
    NBUF = 5
    assert chunks % NBUF == 0
    mesh = plsc.VectorSubcoreMesh(core_axis_name="c", subcore_axis_name="s")

    @functools.partial(
        pl.kernel,
        out_type=jax.ShapeDtypeStruct((N, D), jnp.float32),
        mesh=mesh,
        scratch_types=(
            [pltpu.VMEM((chunks, C), jnp.int32)] * 2
            + [pltpu.VMEM((C, D), jnp.float32)] * NBUF
            + [pltpu.VMEM_SHARED((NS * ((3 * L + NS * 8 - 1) // (NS * 8)) * 8,
                                  D), jnp.float32)]
            + [pltpu.SemaphoreType.DMA] * (3 * NBUF)
        ),
    )
    def k(seq_hbm, lbl_hbm, tok_hbm, combo_hbm, out_hbm, *sc):
        sidx_all, cidx_all = sc[0], sc[1]
        tok = sc[2:2 + NBUF]
        combo_spm = sc[2 + NBUF]
        sems = sc[3 + NBUF:]
        tsem, csem, wsem = sems[:NBUF], sems[NBUF:2 * NBUF], sems[2 * NBUF:]
        wid = lax.axis_index("s") * NC + lax.axis_index("c")
        tile_base = wid * rows_per_w
        sid = lax.axis_index("s")
        rows_per_tile = combo_spm.shape[0] // NS
        pltpu.sync_copy(combo_hbm.at[pl.ds(sid * rows_per_tile, rows_per_tile)],
                        combo_spm.at[pl.ds(sid * rows_per_tile, rows_per_tile)])
        pltpu.sync_copy(seq_hbm.at[wid], sidx_all)
        pltpu.sync_copy(lbl_hbm.at[wid], cidx_all)

        def cvt(c, lpos0):
            for j in range(C // _LANES):
                v = lpos0 + (j * _LANES + lax.iota(jnp.int32, _LANES))
                lpos = jnp.where(v >= L, v - L, v)
                sl = pl.ds(j * _LANES, _LANES)
                cidx_all[c, sl] = 3 * lpos + cidx_all[c, sl]
            nxt = lpos0 + C
            return jnp.where(nxt >= L, nxt - L, nxt)

        lax.fori_loop(0, chunks, cvt, jnp.int32(0), unroll=False)
        plsc.subcore_barrier()

        def body(kk, carry):
            c0 = kk * NBUF
            tgs = [pltpu.async_copy(tok_hbm.at[sidx_all.at[c0 + b]],
                                    tok[b], tsem[b]) for b in range(NBUF)]
            cgs = []
            for b in range(NBUF):
                tgs[b].wait()
                cgs.append(pltpu.async_copy(combo_spm.at[cidx_all.at[c0 + b]],
                                            tok[b], csem[b], add=True))
            wbs = []
            for b in range(NBUF):
                cgs[b].wait()
                base = tile_base + (c0 + b) * C
                wbs.append(pltpu.async_copy(tok[b],
                                            out_hbm.at[pl.ds(base, C)],
                                            wsem[b]))
            for b in range(NBUF):
                wbs[b].wait()
            return carry

        lax.fori_loop(0, chunks // NBUF, body, jnp.int32(0), unroll=False)

    seq3 = seq_flat.reshape(NW, chunks, C)
    lbl3 = lbl_flat.reshape(NW, chunks, C)
    return k(seq3, lbl3, token_table, combo)


def kernel(sequence, segment_label, token_table, segment_table):
    B, L = sequence.shape
    D = token_table.shape[1]
    NS = plsc.get_sparse_core_info().num_subcores
    combo_rows = NS * ((3 * L + NS * 8 - 1) // (NS * 8)) * 8
    combo = _build_combo(segment_table, combo_rows)
    seq_flat = sequence.reshape(-1).astype(jnp.int32)
    lbl_flat = segment_label.reshape(-1).astype(jnp.int32)
    out = _sc_lookup(seq_flat, lbl_flat, token_table, combo, L)
    return out.reshape(B, L, D)


# speedup vs baseline: 3.0670x; 1.0886x over previous
import functools
import math

import jax
import jax.numpy as jnp
from jax import lax
from jax.experimental import pallas as pl
from jax.experimental.pallas import tpu as pltpu
from jax.experimental.pallas import tpu_sc as plsc

_LANES = 16


def _combo_tc_body(seg_ref, out_ref):
    R, D = out_ref.shape
    r = lax.broadcasted_iota(jnp.int32, (R, D), 0)
    dcol = lax.broadcasted_iota(jnp.int32, (R, D), 1)
    l3 = r // 3
    s = r - 3 * l3
    half = (dcol // 2).astype(jnp.float32)
    div = jnp.exp(half * (-2.0 * math.log(10000.0) / D))
    ang = l3.astype(jnp.float32) * div
    pe = jnp.where(dcol % 2 == 0, jnp.sin(ang), jnp.cos(ang))
    st = seg_ref[...]
    seg0 = jnp.broadcast_to(st[0:1, :], (R, D))
    seg1 = jnp.broadcast_to(st[1:2, :], (R, D))
    seg2 = jnp.broadcast_to(st[2:3, :], (R, D))
    out_ref[...] = pe + jnp.where(s == 0, seg0, jnp.where(s == 1, seg1, seg2))


def _build_combo(segment_table, rows):
    return pl.pallas_call(
        _combo_tc_body,
        out_shape=jax.ShapeDtypeStruct((rows, segment_table.shape[1]), jnp.float32),
    )(segment_table)


def _sc_lookup(seq_flat, lbl_flat, token_table, combo, L):
    N = seq_flat.shape[0]
    D = token_table.shape[1]
    info = plsc.get_sparse_core_info()
    NC, NS = info.num_cores, info.num_subcores
    NW = NC * NS
    C = 128
    assert N % (NW * C) == 0 and D % _LANES == 0
    rows_per_w = N // NW
    chunks = rows_per_w // C
    assert rows_per_w % L == 0 and C <= L
    NBUF = 5
    assert chunks % NBUF == 0
    mesh = plsc.VectorSubcoreMesh(core_axis_name="c", subcore_axis_name="s")

    @functools.partial(
        pl.kernel,
        out_type=jax.ShapeDtypeStruct((N, D), jnp.float32),
        mesh=mesh,
        scratch_types=(
            [pltpu.VMEM((chunks, C), jnp.int32)] * 2
            + [pltpu.VMEM((C, D), jnp.float32)] * NBUF
            + [pltpu.VMEM_SHARED((NS * ((3 * L + NS * 8 - 1) // (NS * 8)) * 8,
                                  D), jnp.float32)]
            + [pltpu.SemaphoreType.DMA] * (3 * NBUF)
        ),
    )
    def k(seq_hbm, lbl_hbm, tok_hbm, combo_hbm, out_hbm, *sc):
        sidx_all, cidx_all = sc[0], sc[1]
        tok = sc[2:2 + NBUF]
        combo_spm = sc[2 + NBUF]
        sems = sc[3 + NBUF:]
        tsem, csem, wsem = sems[:NBUF], sems[NBUF:2 * NBUF], sems[2 * NBUF:]
        wid = lax.axis_index("s") * NC + lax.axis_index("c")
        tile_base = wid * rows_per_w
        sid = lax.axis_index("s")
        rows_per_tile = combo_spm.shape[0] // NS
        pltpu.sync_copy(combo_hbm.at[pl.ds(sid * rows_per_tile, rows_per_tile)],
                        combo_spm.at[pl.ds(sid * rows_per_tile, rows_per_tile)])
        pltpu.sync_copy(seq_hbm.at[wid], sidx_all)
        pltpu.sync_copy(lbl_hbm.at[wid], cidx_all)

        def cvt(c, lpos0):
            for j in range(C // _LANES):
                v = lpos0 + (j * _LANES + lax.iota(jnp.int32, _LANES))
                lpos = jnp.where(v >= L, v - L, v)
                sl = pl.ds(j * _LANES, _LANES)
                cidx_all[c, sl] = 3 * lpos + cidx_all[c, sl]
            nxt = lpos0 + C
            return jnp.where(nxt >= L, nxt - L, nxt)

        lax.fori_loop(0, chunks, cvt, jnp.int32(0), unroll=False)
        plsc.subcore_barrier()

        def body(kk, carry):
            c0 = kk * NBUF
            tgs = []
            for b in range(NBUF):
                @pl.when(kk > 0)
                def _(b=b):
                    pltpu.make_async_copy(
                        tok[b], out_hbm.at[pl.ds(tile_base, C)], wsem[b]
                    ).wait()
                tgs.append(pltpu.async_copy(tok_hbm.at[sidx_all.at[c0 + b]],
                                            tok[b], tsem[b]))
            cgs = []
            for b in range(NBUF):
                tgs[b].wait()
                cgs.append(pltpu.async_copy(combo_spm.at[cidx_all.at[c0 + b]],
                                            tok[b], csem[b], add=True))
            for b in range(NBUF):
                cgs[b].wait()
                base = tile_base + (c0 + b) * C
                pltpu.async_copy(tok[b], out_hbm.at[pl.ds(base, C)], wsem[b])
            return carry

        lax.fori_loop(0, chunks // NBUF, body, jnp.int32(0), unroll=False)
        for b in range(NBUF):
            pltpu.make_async_copy(
                tok[b], out_hbm.at[pl.ds(tile_base, C)], wsem[b]
            ).wait()

    seq3 = seq_flat.reshape(NW, chunks, C)
    lbl3 = lbl_flat.reshape(NW, chunks, C)
    return k(seq3, lbl3, token_table, combo)


def kernel(sequence, segment_label, token_table, segment_table):
    B, L = sequence.shape
    D = token_table.shape[1]
    NS = plsc.get_sparse_core_info().num_subcores
    combo_rows = NS * ((3 * L + NS * 8 - 1) // (NS * 8)) * 8
    combo = _build_combo(segment_table, combo_rows)
    seq_flat = sequence.reshape(-1).astype(jnp.int32)
    lbl_flat = segment_label.reshape(-1).astype(jnp.int32)
    out = _sc_lookup(seq_flat, lbl_flat, token_table, combo, L)
    return out.reshape(B, L, D)
